# baseline (speedup 1.0000x reference)
import functools

import jax
import jax.numpy as jnp
from jax import lax
from jax.experimental import pallas as pl
from jax.experimental.pallas import tpu as pltpu
from jax.experimental.pallas import tpu_sc as plsc

D = 1024
E = 8
H = 2048
T = 2048
M = 256
NT = T // M + (E - 1)
P = NT * M
NTP = 128


def _route_body(x_ref, gw_ref, gb_ref, pos_ref, te_ref):
    xs = x_ref[...]
    gw = gw_ref[...]
    scoresT = lax.dot_general(gw, xs, (((0,), (1,)), ((), ())),
                              preferred_element_type=jnp.float32)
    scoresT = scoresT + gb_ref[...].reshape(E, 1)

    eids = lax.broadcasted_iota(jnp.int32, (E, T), 0)
    mx = jnp.max(scoresT, axis=0, keepdims=True)
    selT = jnp.min(jnp.where(scoresT == mx, eids, E), axis=0, keepdims=True)
    oh = (eids == selT).astype(jnp.int32)

    c = oh
    k = 1
    while k < T:
        c = c + jnp.concatenate(
            [jnp.zeros((E, k), jnp.int32), c[:, : T - k]], axis=1)
        k *= 2

    counts = c[:, T - 1 : T]
    caps = jnp.bitwise_and(counts + (M - 1), -M)
    ic = caps
    k = 1
    while k < E:
        ic = ic + jnp.concatenate(
            [jnp.zeros((k, 1), jnp.int32), ic[: E - k, :]], axis=0)
        k *= 2
    offs = ic - caps

    rank = jnp.sum(c * oh, axis=0, keepdims=True) - 1
    base = jnp.sum(oh * offs, axis=0, keepdims=True)
    pos_ref[...] = base + rank

    lane = lax.broadcasted_iota(jnp.int32, (E, NTP), 1)
    lo = jnp.right_shift(offs, 8)
    hi = jnp.right_shift(ic, 8)
    te_ref[...] = (jnp.where(lane == 0, jnp.broadcast_to(lo, (E, NTP)), 0)
                   + jnp.where(lane == 1, jnp.broadcast_to(hi, (E, NTP)), 0))


def _route(x2, gate_w, gate_b2):
    return pl.pallas_call(
        _route_body,
        out_shape=(
            jax.ShapeDtypeStruct((1, T), jnp.int32),
            jax.ShapeDtypeStruct((E, NTP), jnp.int32),
        ),
    )(x2, gate_w, gate_b2)


_NC = 2
_NS = 16
_NW = _NC * _NS
ROWS_W = T // _NW


@functools.lru_cache(maxsize=None)
def _sc_kernels():
    mesh = plsc.VectorSubcoreMesh(
        core_axis_name="c", subcore_axis_name="s",
        num_cores=_NC, num_subcores=_NS)
    @functools.partial(
        pl.kernel,
        out_type=jax.ShapeDtypeStruct((P, D), jnp.float32),
        mesh=mesh,
        scratch_types=[
            pltpu.VMEM((ROWS_W,), jnp.int32),
            pltpu.VMEM((ROWS_W, D), jnp.float32),
            pltpu.SemaphoreType.DMA,
        ],
    )
    def dispatch_k(x_hbm, pos_hbm, xs_hbm, idx_v, rows_v, sem):
        wid = lax.axis_index("s") * _NC + lax.axis_index("c")
        rbase = wid * ROWS_W
        pltpu.sync_copy(pos_hbm.at[pl.ds(rbase, ROWS_W)], idx_v)
        pltpu.sync_copy(x_hbm.at[pl.ds(rbase, ROWS_W)], rows_v)
        pltpu.async_copy(rows_v, xs_hbm.at[idx_v], sem).wait()

    @functools.partial(
        pl.kernel,
        out_type=jax.ShapeDtypeStruct((T, D), jnp.float32),
        mesh=mesh,
        scratch_types=[
            pltpu.VMEM((ROWS_W,), jnp.int32),
            pltpu.VMEM((ROWS_W, D), jnp.float32),
            pltpu.SemaphoreType.DMA,
        ],
    )
    def combine_k(ys_hbm, pos_hbm, out_hbm, idx_v, rows_v, sem):
        wid = lax.axis_index("s") * _NC + lax.axis_index("c")
        rbase = wid * ROWS_W
        pltpu.sync_copy(pos_hbm.at[pl.ds(rbase, ROWS_W)], idx_v)
        pltpu.async_copy(ys_hbm.at[idx_v], rows_v, sem).wait()
        pltpu.sync_copy(rows_v, out_hbm.at[pl.ds(rbase, ROWS_W)])

    return dispatch_k, combine_k


NBUF = 4
HC = 4
HCS = 2
HB = H // HC
NCH = E * HC


def _mlp_body(sp_ref, xs_ref, w1_hbm, b1_ref, w2_hbm, b2_ref, ys_ref,
              w1r, w2r, sems):
    def w_copies(c, slot):
        e = c >> HCS
        j = c & (HC - 1)
        return (
            pltpu.make_async_copy(w1_hbm.at[e, :, pl.ds(j * HB, HB)],
                                  w1r.at[slot], sems.at[0, slot]),
            pltpu.make_async_copy(w2_hbm.at[e, pl.ds(j * HB, HB), :],
                                  w2r.at[slot], sems.at[1, slot]),
        )

    for c in range(NBUF):
        for cp in w_copies(c, c):
            cp.start()

    def chunk_body(c, _):
        slot = lax.rem(c, NBUF)
        e = c >> HCS
        j = c & (HC - 1)
        c1, c2 = w_copies(c, slot)
        c1.wait()
        c2.wait()
        w1 = w1r[slot].astype(jnp.bfloat16)
        w2 = w2r[slot].astype(jnp.bfloat16)
        b1v = b1_ref[e, :, pl.ds(j * HB, HB)]
        b2v = b2_ref[e, :, :]
        lo = sp_ref[e]
        hi = sp_ref[E + e]

        def tile_body(t, _):
            r = t * M
            xt = xs_ref[pl.ds(r, M), :].astype(jnp.bfloat16)
            h = jnp.dot(xt, w1, preferred_element_type=jnp.float32)
            h = h + b1v
            h = 0.5 * h * (1.0 + lax.erf(h * 0.7071067811865476))
            o = jnp.dot(h.astype(jnp.bfloat16), w2,
                        preferred_element_type=jnp.float32)

            @pl.when(j == 0)
            def _():
                ys_ref[pl.ds(r, M), :] = o + b2v

            @pl.when(j != 0)
            def _():
                ys_ref[pl.ds(r, M), :] += o

            return 0

        lax.fori_loop(lo, hi, tile_body, 0)

        nxt = c + NBUF

        @pl.when(nxt < NCH)
        def _():
            for cp in w_copies(nxt, slot):
                cp.start()

        return 0

    lax.fori_loop(0, NCH, chunk_body, 0)


def _mlp(sp, xs, W1, b1, W2, b2):
    grid_spec = pltpu.PrefetchScalarGridSpec(
        num_scalar_prefetch=1,
        grid=(1,),
        in_specs=[
            pl.BlockSpec((P, D), lambda i, sp: (0, 0)),
            pl.BlockSpec(memory_space=pl.ANY),
            pl.BlockSpec((E, 1, H), lambda i, sp: (0, 0, 0)),
            pl.BlockSpec(memory_space=pl.ANY),
            pl.BlockSpec((E, 1, D), lambda i, sp: (0, 0, 0)),
        ],
        out_specs=pl.BlockSpec((P, D), lambda i, sp: (0, 0)),
        scratch_shapes=[
            pltpu.VMEM((NBUF, D, HB), jnp.float32),
            pltpu.VMEM((NBUF, HB, D), jnp.float32),
            pltpu.SemaphoreType.DMA((2, NBUF)),
        ],
    )
    return pl.pallas_call(
        _mlp_body,
        grid_spec=grid_spec,
        out_shape=jax.ShapeDtypeStruct((P, D), jnp.float32),
        compiler_params=pltpu.CompilerParams(
            dimension_semantics=("arbitrary",),
            vmem_limit_bytes=128 * 1024 * 1024,
        ),
    )(sp, xs, W1, b1, W2, b2)


def kernel(x, gate_w, gate_b, W1, b1, W2, b2):
    B, S, _ = x.shape
    x2 = x.reshape(T, D)
    pos2, te2 = _route(x2, gate_w, gate_b.reshape(1, E))
    pos = pos2.reshape(T)
    sp = jnp.concatenate([te2[:, 0], te2[:, 1]])
    dispatch_k, combine_k = _sc_kernels()
    xs = dispatch_k(x2, pos)
    ys = _mlp(sp, xs, W1, b1.reshape(E, 1, H), W2, b2.reshape(E, 1, D))
    out = combine_k(ys, pos)
    return out.reshape(B, S, D), jnp.zeros((), jnp.float32)


# speedup vs baseline: 1.1459x; 1.1459x over previous
import functools

import jax
import jax.numpy as jnp
from jax import lax
from jax.experimental import pallas as pl
from jax.experimental.pallas import tpu as pltpu
from jax.experimental.pallas import tpu_sc as plsc

D = 1024
E = 8
H = 2048
T = 2048
M = 256
NT = T // M + (E - 1)
P = NT * M
NTP = 128


def _route_body(x_ref, gw_ref, gb_ref, pos_ref, te_ref):
    xs = x_ref[...]
    gw = gw_ref[...]
    scoresT = lax.dot_general(gw, xs, (((0,), (1,)), ((), ())),
                              preferred_element_type=jnp.float32)
    scoresT = scoresT + gb_ref[...].reshape(E, 1)

    eids = lax.broadcasted_iota(jnp.int32, (E, T), 0)
    mx = jnp.max(scoresT, axis=0, keepdims=True)
    selT = jnp.min(jnp.where(scoresT == mx, eids, E), axis=0, keepdims=True)
    oh = (eids == selT).astype(jnp.int32)

    c = oh
    k = 1
    while k < T:
        c = c + jnp.concatenate(
            [jnp.zeros((E, k), jnp.int32), c[:, : T - k]], axis=1)
        k *= 2

    counts = c[:, T - 1 : T]
    caps = jnp.bitwise_and(counts + (M - 1), -M)
    ic = caps
    k = 1
    while k < E:
        ic = ic + jnp.concatenate(
            [jnp.zeros((k, 1), jnp.int32), ic[: E - k, :]], axis=0)
        k *= 2
    offs = ic - caps

    rank = jnp.sum(c * oh, axis=0, keepdims=True) - 1
    base = jnp.sum(oh * offs, axis=0, keepdims=True)
    pos_ref[...] = base + rank

    lane = lax.broadcasted_iota(jnp.int32, (E, NTP), 1)
    lo = jnp.right_shift(offs, 8)
    hi = jnp.right_shift(ic, 8)
    te_ref[...] = (jnp.where(lane == 0, jnp.broadcast_to(lo, (E, NTP)), 0)
                   + jnp.where(lane == 1, jnp.broadcast_to(hi, (E, NTP)), 0))


def _route(x2, gate_w, gate_b2):
    return pl.pallas_call(
        _route_body,
        out_shape=(
            jax.ShapeDtypeStruct((1, T), jnp.int32),
            jax.ShapeDtypeStruct((E, NTP), jnp.int32),
        ),
    )(x2, gate_w, gate_b2)


_NC = 2
_NS = 16
_NW = _NC * _NS
ROWS_W = T // _NW


@functools.lru_cache(maxsize=None)
def _sc_kernels():
    mesh = plsc.VectorSubcoreMesh(
        core_axis_name="c", subcore_axis_name="s",
        num_cores=_NC, num_subcores=_NS)
    @functools.partial(
        pl.kernel,
        out_type=jax.ShapeDtypeStruct((P, D), jnp.float32),
        mesh=mesh,
        scratch_types=[
            pltpu.VMEM((ROWS_W,), jnp.int32),
            pltpu.VMEM((ROWS_W, D), jnp.float32),
            pltpu.SemaphoreType.DMA,
        ],
    )
    def dispatch_k(x_hbm, pos_hbm, xs_hbm, idx_v, rows_v, sem):
        wid = lax.axis_index("s") * _NC + lax.axis_index("c")
        rbase = wid * ROWS_W
        pltpu.sync_copy(pos_hbm.at[pl.ds(rbase, ROWS_W)], idx_v)
        pltpu.sync_copy(x_hbm.at[pl.ds(rbase, ROWS_W)], rows_v)
        pltpu.async_copy(rows_v, xs_hbm.at[idx_v], sem).wait()

    @functools.partial(
        pl.kernel,
        out_type=jax.ShapeDtypeStruct((T, D), jnp.float32),
        mesh=mesh,
        scratch_types=[
            pltpu.VMEM((ROWS_W,), jnp.int32),
            pltpu.VMEM((ROWS_W, D), jnp.float32),
            pltpu.SemaphoreType.DMA,
        ],
    )
    def combine_k(ys_hbm, pos_hbm, out_hbm, idx_v, rows_v, sem):
        wid = lax.axis_index("s") * _NC + lax.axis_index("c")
        rbase = wid * ROWS_W
        pltpu.sync_copy(pos_hbm.at[pl.ds(rbase, ROWS_W)], idx_v)
        pltpu.async_copy(ys_hbm.at[idx_v], rows_v, sem).wait()
        pltpu.sync_copy(rows_v, out_hbm.at[pl.ds(rbase, ROWS_W)])

    return dispatch_k, combine_k


NBUF = 3
HC = 2
HCS = 1
HB = H // HC
NCH = E * HC


def _mlp_body(sp_ref, xs_ref, w1_hbm, b1_ref, w2_hbm, b2_ref, ys_ref,
              w1r, w2r, sems):
    def w_copies(c, slot):
        e = c >> HCS
        j = c & (HC - 1)
        return (
            pltpu.make_async_copy(w1_hbm.at[e, :, pl.ds(j * HB, HB)],
                                  w1r.at[slot], sems.at[0, slot]),
            pltpu.make_async_copy(w2_hbm.at[e, pl.ds(j * HB, HB), :],
                                  w2r.at[slot], sems.at[1, slot]),
        )

    for c in range(NBUF):
        for cp in w_copies(c, c):
            cp.start()

    def chunk_body(c, _):
        slot = lax.rem(c, NBUF)
        e = c >> HCS
        j = c & (HC - 1)
        c1, c2 = w_copies(c, slot)
        c1.wait()
        c2.wait()
        w1 = w1r[slot].astype(jnp.bfloat16)
        w2 = w2r[slot].astype(jnp.bfloat16)
        b1v = b1_ref[e, :, pl.ds(j * HB, HB)]
        b2v = b2_ref[e, :, :]
        lo = sp_ref[e]
        hi = sp_ref[E + e]

        def tile_body(t, _):
            r = t * M
            xt = xs_ref[pl.ds(r, M), :].astype(jnp.bfloat16)
            h = jnp.dot(xt, w1, preferred_element_type=jnp.float32)
            h = h + b1v
            h = 0.5 * h * (1.0 + lax.erf(h * 0.7071067811865476))
            o = jnp.dot(h.astype(jnp.bfloat16), w2,
                        preferred_element_type=jnp.float32)

            @pl.when(j == 0)
            def _():
                ys_ref[pl.ds(r, M), :] = o + b2v

            @pl.when(j != 0)
            def _():
                ys_ref[pl.ds(r, M), :] += o

            return 0

        lax.fori_loop(lo, hi, tile_body, 0)

        nxt = c + NBUF

        @pl.when(nxt < NCH)
        def _():
            for cp in w_copies(nxt, slot):
                cp.start()

        return 0

    lax.fori_loop(0, NCH, chunk_body, 0)


def _mlp(sp, xs, W1, b1, W2, b2):
    grid_spec = pltpu.PrefetchScalarGridSpec(
        num_scalar_prefetch=1,
        grid=(1,),
        in_specs=[
            pl.BlockSpec((P, D), lambda i, sp: (0, 0)),
            pl.BlockSpec(memory_space=pl.ANY),
            pl.BlockSpec((E, 1, H), lambda i, sp: (0, 0, 0)),
            pl.BlockSpec(memory_space=pl.ANY),
            pl.BlockSpec((E, 1, D), lambda i, sp: (0, 0, 0)),
        ],
        out_specs=pl.BlockSpec((P, D), lambda i, sp: (0, 0)),
        scratch_shapes=[
            pltpu.VMEM((NBUF, D, HB), jnp.float32),
            pltpu.VMEM((NBUF, HB, D), jnp.float32),
            pltpu.SemaphoreType.DMA((2, NBUF)),
        ],
    )
    return pl.pallas_call(
        _mlp_body,
        grid_spec=grid_spec,
        out_shape=jax.ShapeDtypeStruct((P, D), jnp.float32),
        compiler_params=pltpu.CompilerParams(
            dimension_semantics=("arbitrary",),
            vmem_limit_bytes=128 * 1024 * 1024,
        ),
    )(sp, xs, W1, b1, W2, b2)


def kernel(x, gate_w, gate_b, W1, b1, W2, b2):
    B, S, _ = x.shape
    x2 = x.reshape(T, D)
    pos2, te2 = _route(x2, gate_w, gate_b.reshape(1, E))
    pos = pos2.reshape(T)
    sp = jnp.concatenate([te2[:, 0], te2[:, 1]])
    dispatch_k, combine_k = _sc_kernels()
    xs = dispatch_k(x2, pos)
    ys = _mlp(sp, xs, W1, b1.reshape(E, 1, H), W2, b2.reshape(E, 1, D))
    out = combine_k(ys, pos)
    return out.reshape(B, S, D), jnp.zeros((), jnp.float32)
